# baseline (speedup 1.0000x reference)
import functools

import jax
import jax.numpy as jnp
from jax.experimental import pallas as pl
from jax.experimental.pallas import tpu as pltpu

H = 64
W_IMG = 64
C = 21
N = H * W_IMG
THETA_ALPHA = 80.0
THETA_BETA = 13.0
THETA_GAMMA = 3.0
N_ITERS = 5

BM = 512
NB = N // BM
WIN_ROWS = 44
WIN = WIN_ROWS * W_IMG


def _win_start(j):
    start = jnp.clip(4 * j - 9, 0, (H - WIN_ROWS) // 2) * 128
    return pl.multiple_of(start, 128)


def _ks_build_kernel(fsz_ref, ft_ref, sqsh_r_ref, sqsh_c_ref, ks_ref):
    j = pl.program_id(0)
    start = _win_start(j)
    fz = fsz_ref[pl.ds(start, WIN), :]
    ftc = ft_ref[:, pl.ds(j * BM, BM)]
    g = jnp.dot(fz, ftc, preferred_element_type=jnp.float32)
    arg = sqsh_r_ref[pl.ds(start, WIN), :] + sqsh_c_ref[:, pl.ds(j * BM, BM)] + g
    k = jnp.minimum(jnp.exp(arg), 1.0)
    rows = start + jax.lax.broadcasted_iota(jnp.int32, (WIN, BM), 0)
    cols = j * BM + jax.lax.broadcasted_iota(jnp.int32, (WIN, BM), 1)
    ks_ref[...] = jnp.where(rows == cols, 0.0, k).astype(jnp.bfloat16)


def _fused_kernel(fbz_ref, ft_ref, sqbh_r_ref, sqbh_c_ref, ut_ref, ks_ref,
                  wb_ref, ws_ref, out_ref, kb_s, qa_s, qb_s):
    t = pl.program_id(0)
    j = pl.program_id(1)
    tile = jax.lax.rem(j + NB - 1, NB)
    it = t + jnp.where(j > 0, 1, 0)
    col = pl.multiple_of(tile * BM, BM)

    def _mf_step(src_s, dst_s):
        qtb = src_s[...]
        qwin = src_s[:, pl.ds(_win_start(tile), WIN)]
        kb_col = kb_s[:, pl.ds(col, BM)]
        mb = jnp.dot(qtb, kb_col, preferred_element_type=jnp.float32)
        ms = jnp.dot(qwin, ks_ref[...], preferred_element_type=jnp.float32)
        msg = wb_ref[0] * mb + ws_ref[0] * ms
        pw = jnp.sum(msg, axis=0, keepdims=True) - msg
        logits = -ut_ref[:, pl.ds(col, BM)] - pw
        m = jnp.max(logits, axis=0, keepdims=True)
        e = jnp.exp(logits - m)
        q = e / jnp.sum(e, axis=0, keepdims=True)
        dst_s[:, pl.ds(col, BM)] = q.astype(jnp.bfloat16)

        @pl.when(it == N_ITERS)
        def _emit():
            out_ref[:, pl.ds(col, BM)] = q

    do_dot = jnp.logical_not((t == 0) & (j == 0)) & (it <= N_ITERS)

    @pl.when(do_dot & (jax.lax.rem(it, 2) == 1))
    def _dot_odd():
        _mf_step(qa_s, qb_s)

    @pl.when(do_dot & (jax.lax.rem(it, 2) == 0))
    def _dot_even():
        _mf_step(qb_s, qa_s)

    @pl.when(t == 0)
    def _build():
        fz = fbz_ref[pl.ds(j * BM, BM), :]
        ft = ft_ref[...]
        g = jnp.dot(fz, ft, preferred_element_type=jnp.float32)
        arg = sqbh_r_ref[pl.ds(j * BM, BM), :] + sqbh_c_ref[...] + g
        kb = jnp.minimum(jnp.exp(arg), 1.0)
        rows = j * BM + jax.lax.broadcasted_iota(jnp.int32, (BM, N), 0)
        cols = jax.lax.broadcasted_iota(jnp.int32, (BM, N), 1)
        kb_bf = jnp.where(rows == cols, 0.0, kb).astype(jnp.bfloat16)
        kb_s[pl.ds(j * BM, BM), :] = kb_bf

    @pl.when((t == 0) & (j == 0))
    def _init_q():
        logits = -ut_ref[...]
        m = jnp.max(logits, axis=0, keepdims=True)
        e = jnp.exp(logits - m)
        q0 = e / jnp.sum(e, axis=0, keepdims=True)
        qa_s[...] = q0.astype(jnp.bfloat16)


@functools.partial(jax.jit, static_argnames=())
def kernel(unary, image, w_bilateral, w_spatial):
    h, w, c = unary.shape
    n = h * w
    UT = unary.reshape(n, c).T

    ys, xs = jnp.meshgrid(jnp.arange(h, dtype=jnp.float32),
                          jnp.arange(w, dtype=jnp.float32), indexing="ij")
    coords = jnp.stack([xs, ys], axis=-1).reshape(-1, 2)
    rgb = image.reshape(-1, 3)
    fb = jnp.concatenate([coords / THETA_ALPHA, rgb / THETA_BETA], axis=-1)
    fs = coords / THETA_GAMMA
    zeros3 = jnp.zeros((n, 3), jnp.float32)
    zeros5 = jnp.zeros((n, 5), jnp.float32)
    fbz = jnp.concatenate([fb, zeros3], axis=-1)
    fsz = jnp.concatenate([zeros5, fs, zeros3[:, :1]], axis=-1)
    ft = (fbz + fsz).T
    sqbh = -0.5 * jnp.sum(fb * fb, axis=-1)
    sqsh = -0.5 * jnp.sum(fs * fs, axis=-1)
    wb = jnp.reshape(w_bilateral.astype(jnp.float32), (1,))
    ws = jnp.reshape(w_spatial.astype(jnp.float32), (1,))

    Ks = pl.pallas_call(
        _ks_build_kernel,
        grid=(NB,),
        in_specs=[
            pl.BlockSpec((n, 8), lambda j: (0, 0)),
            pl.BlockSpec((8, n), lambda j: (0, 0)),
            pl.BlockSpec((n, 1), lambda j: (0, 0)),
            pl.BlockSpec((1, n), lambda j: (0, 0)),
        ],
        out_specs=pl.BlockSpec((WIN, BM), lambda j: (0, j)),
        out_shape=jax.ShapeDtypeStruct((WIN, n), jnp.bfloat16),
    )(fsz, ft, sqsh[:, None], sqsh[None, :])

    QT = pl.pallas_call(
        _fused_kernel,
        grid=(N_ITERS + 1, NB),
        in_specs=[
            pl.BlockSpec((n, 8), lambda t, j: (0, 0)),
            pl.BlockSpec((8, n), lambda t, j: (0, 0)),
            pl.BlockSpec((n, 1), lambda t, j: (0, 0)),
            pl.BlockSpec((1, n), lambda t, j: (0, 0)),
            pl.BlockSpec((c, n), lambda t, j: (0, 0)),
            pl.BlockSpec((WIN, BM), lambda t, j: (0, jax.lax.rem(j + NB - 1, NB))),
            pl.BlockSpec(memory_space=pltpu.SMEM),
            pl.BlockSpec(memory_space=pltpu.SMEM),
        ],
        out_specs=pl.BlockSpec((c, n), lambda t, j: (0, 0)),
        out_shape=jax.ShapeDtypeStruct((c, n), jnp.float32),
        scratch_shapes=[
            pltpu.VMEM((n, n), jnp.bfloat16),
            pltpu.VMEM((c, n), jnp.bfloat16),
            pltpu.VMEM((c, n), jnp.bfloat16),
        ],
        compiler_params=pltpu.CompilerParams(
            vmem_limit_bytes=63 * 1024 * 1024,
        ),
    )(fbz, ft, sqbh[:, None], sqbh[None, :], UT, Ks, wb, ws)
    return QT.T.reshape(h, w, c)


# speedup vs baseline: 1.0759x; 1.0759x over previous
import functools

import jax
import jax.numpy as jnp
from jax.experimental import pallas as pl
from jax.experimental.pallas import tpu as pltpu

H = 64
W_IMG = 64
C = 21
N = H * W_IMG
THETA_ALPHA = 80.0
THETA_BETA = 13.0
THETA_GAMMA = 3.0
N_ITERS = 5

BM = 512
NB = N // BM
WIN_ROWS = 40
WIN = WIN_ROWS * W_IMG


def _win_start(j):
    start = jnp.clip(4 * j - 8, 0, (H - WIN_ROWS) // 2) * 128
    return pl.multiple_of(start, 128)


def _ks_build_kernel(fsz_ref, ft_ref, sqsh_r_ref, sqsh_c_ref, ks_ref):
    j = pl.program_id(0)
    start = _win_start(j)
    fz = fsz_ref[pl.ds(start, WIN), :]
    ftc = ft_ref[:, pl.ds(j * BM, BM)]
    g = jnp.dot(fz, ftc, preferred_element_type=jnp.float32)
    arg = sqsh_r_ref[pl.ds(start, WIN), :] + sqsh_c_ref[:, pl.ds(j * BM, BM)] + g
    k = jnp.minimum(jnp.exp(arg), 1.0)
    rows = start + jax.lax.broadcasted_iota(jnp.int32, (WIN, BM), 0)
    cols = j * BM + jax.lax.broadcasted_iota(jnp.int32, (WIN, BM), 1)
    ks_ref[...] = jnp.where(rows == cols, 0.0, k).astype(jnp.bfloat16)


def _fused_kernel(fbz_ref, ft_ref, sqbh_r_ref, sqbh_c_ref, ut_ref, ks_ref,
                  wb_ref, ws_ref, out_ref, kb_s, qa_s, qb_s):
    t = pl.program_id(0)
    j = pl.program_id(1)

    def _mf_step(src_s, dst_s):
        qtb = src_s[...]
        qwin = src_s[:, pl.ds(_win_start(j), WIN)]
        kb_col = kb_s[:, pl.ds(j * BM, BM)]
        mb = jnp.dot(qtb, kb_col, preferred_element_type=jnp.float32)
        ms = jnp.dot(qwin, ks_ref[...], preferred_element_type=jnp.float32)
        msg = wb_ref[0] * mb + ws_ref[0] * ms
        pw = jnp.sum(msg, axis=0, keepdims=True) - msg
        logits = -ut_ref[:, pl.ds(j * BM, BM)] - pw
        m = jnp.max(logits, axis=0, keepdims=True)
        e = jnp.exp(logits - m)
        q = e / jnp.sum(e, axis=0, keepdims=True)
        dst_s[:, pl.ds(j * BM, BM)] = q.astype(jnp.bfloat16)

        @pl.when(t == N_ITERS)
        def _emit():
            out_ref[:, pl.ds(j * BM, BM)] = q

    @pl.when(t == 0)
    def _build():
        fz = fbz_ref[pl.ds(j * BM, BM), :]
        ft = ft_ref[...]
        g = jnp.dot(fz, ft, preferred_element_type=jnp.float32)
        arg = sqbh_r_ref[pl.ds(j * BM, BM), :] + sqbh_c_ref[...] + g
        kb = jnp.minimum(jnp.exp(arg), 1.0)
        rows = j * BM + jax.lax.broadcasted_iota(jnp.int32, (BM, N), 0)
        cols = jax.lax.broadcasted_iota(jnp.int32, (BM, N), 1)
        kb_bf = jnp.where(rows == cols, 0.0, kb).astype(jnp.bfloat16)
        kb_s[pl.ds(j * BM, BM), :] = kb_bf

        @pl.when(j == 0)
        def _init_q():
            logits = -ut_ref[...]
            m = jnp.max(logits, axis=0, keepdims=True)
            e = jnp.exp(logits - m)
            q0 = e / jnp.sum(e, axis=0, keepdims=True)
            qa_s[...] = q0.astype(jnp.bfloat16)

    @pl.when((t >= 1) & (t % 2 == 1))
    def _odd():
        _mf_step(qa_s, qb_s)

    @pl.when((t >= 2) & (t % 2 == 0))
    def _even():
        _mf_step(qb_s, qa_s)


@functools.partial(jax.jit, static_argnames=())
def kernel(unary, image, w_bilateral, w_spatial):
    h, w, c = unary.shape
    n = h * w
    UT = unary.reshape(n, c).T

    ys, xs = jnp.meshgrid(jnp.arange(h, dtype=jnp.float32),
                          jnp.arange(w, dtype=jnp.float32), indexing="ij")
    coords = jnp.stack([xs, ys], axis=-1).reshape(-1, 2)
    rgb = image.reshape(-1, 3)
    fb = jnp.concatenate([coords / THETA_ALPHA, rgb / THETA_BETA], axis=-1)
    fs = coords / THETA_GAMMA
    zeros3 = jnp.zeros((n, 3), jnp.float32)
    zeros5 = jnp.zeros((n, 5), jnp.float32)
    fbz = jnp.concatenate([fb, zeros3], axis=-1)
    fsz = jnp.concatenate([zeros5, fs, zeros3[:, :1]], axis=-1)
    ft = (fbz + fsz).T
    sqbh = -0.5 * jnp.sum(fb * fb, axis=-1)
    sqsh = -0.5 * jnp.sum(fs * fs, axis=-1)
    wb = jnp.reshape(w_bilateral.astype(jnp.float32), (1,))
    ws = jnp.reshape(w_spatial.astype(jnp.float32), (1,))

    Ks = pl.pallas_call(
        _ks_build_kernel,
        grid=(NB,),
        in_specs=[
            pl.BlockSpec((n, 8), lambda j: (0, 0)),
            pl.BlockSpec((8, n), lambda j: (0, 0)),
            pl.BlockSpec((n, 1), lambda j: (0, 0)),
            pl.BlockSpec((1, n), lambda j: (0, 0)),
        ],
        out_specs=pl.BlockSpec((WIN, BM), lambda j: (0, j)),
        out_shape=jax.ShapeDtypeStruct((WIN, n), jnp.bfloat16),
    )(fsz, ft, sqsh[:, None], sqsh[None, :])

    QT = pl.pallas_call(
        _fused_kernel,
        grid=(N_ITERS + 1, NB),
        in_specs=[
            pl.BlockSpec((n, 8), lambda t, j: (0, 0)),
            pl.BlockSpec((8, n), lambda t, j: (0, 0)),
            pl.BlockSpec((n, 1), lambda t, j: (0, 0)),
            pl.BlockSpec((1, n), lambda t, j: (0, 0)),
            pl.BlockSpec((c, n), lambda t, j: (0, 0)),
            pl.BlockSpec((WIN, BM), lambda t, j: (0, jnp.where(t == 0, 0, j))),
            pl.BlockSpec(memory_space=pltpu.SMEM),
            pl.BlockSpec(memory_space=pltpu.SMEM),
        ],
        out_specs=pl.BlockSpec((c, n), lambda t, j: (0, 0)),
        out_shape=jax.ShapeDtypeStruct((c, n), jnp.float32),
        scratch_shapes=[
            pltpu.VMEM((n, n), jnp.bfloat16),
            pltpu.VMEM((c, n), jnp.bfloat16),
            pltpu.VMEM((c, n), jnp.bfloat16),
        ],
        compiler_params=pltpu.CompilerParams(
            vmem_limit_bytes=63 * 1024 * 1024,
        ),
    )(fbz, ft, sqbh[:, None], sqbh[None, :], UT, Ks, wb, ws)
    return QT.T.reshape(h, w, c)


# speedup vs baseline: 1.1274x; 1.0479x over previous
import functools

import jax
import jax.numpy as jnp
from jax.experimental import pallas as pl
from jax.experimental.pallas import tpu as pltpu

H = 64
W_IMG = 64
C = 21
N = H * W_IMG
THETA_ALPHA = 80.0
THETA_BETA = 13.0
THETA_GAMMA = 3.0
N_ITERS = 5

BM = 1024
NB = N // BM
WIN_ROWS = 48
WIN = WIN_ROWS * W_IMG


def _win_start(j):
    start = jnp.clip(8 * j - 8, 0, (H - WIN_ROWS) // 2) * 128
    return pl.multiple_of(start, 128)


def _ks_build_kernel(fsz_ref, ft_ref, sqsh_r_ref, sqsh_c_ref, ks_ref):
    j = pl.program_id(0)
    start = _win_start(j)
    fz = fsz_ref[pl.ds(start, WIN), :]
    ftc = ft_ref[:, pl.ds(j * BM, BM)]
    g = jnp.dot(fz, ftc, preferred_element_type=jnp.float32)
    arg = sqsh_r_ref[pl.ds(start, WIN), :] + sqsh_c_ref[:, pl.ds(j * BM, BM)] + g
    k = jnp.minimum(jnp.exp(arg), 1.0)
    rows = start + jax.lax.broadcasted_iota(jnp.int32, (WIN, BM), 0)
    cols = j * BM + jax.lax.broadcasted_iota(jnp.int32, (WIN, BM), 1)
    ks_ref[...] = jnp.where(rows == cols, 0.0, k).astype(jnp.bfloat16)


def _fused_kernel(fbz_ref, ft_ref, sqbh_r_ref, sqbh_c_ref, ut_ref, ks_ref,
                  wb_ref, ws_ref, out_ref, kb_s, qa_s, qb_s):
    t = pl.program_id(0)
    j = pl.program_id(1)

    def _mf_step(src_s, dst_s):
        qtb = src_s[...]
        qwin = src_s[:, pl.ds(_win_start(j), WIN)]
        kb_col = kb_s[:, pl.ds(j * BM, BM)]
        mb = jnp.dot(qtb, kb_col, preferred_element_type=jnp.float32)
        ms = jnp.dot(qwin, ks_ref[...], preferred_element_type=jnp.float32)
        msg = wb_ref[0] * mb + ws_ref[0] * ms
        pw = jnp.sum(msg, axis=0, keepdims=True) - msg
        logits = -ut_ref[:, pl.ds(j * BM, BM)] - pw
        m = jnp.max(logits, axis=0, keepdims=True)
        e = jnp.exp(logits - m)
        q = e / jnp.sum(e, axis=0, keepdims=True)
        dst_s[:, pl.ds(j * BM, BM)] = q.astype(jnp.bfloat16)

        @pl.when(t == N_ITERS)
        def _emit():
            out_ref[:, pl.ds(j * BM, BM)] = q

    @pl.when(t == 0)
    def _build():
        fz = fbz_ref[pl.ds(j * BM, BM), :]
        ft = ft_ref[...]
        g = jnp.dot(fz, ft, preferred_element_type=jnp.float32)
        arg = sqbh_r_ref[pl.ds(j * BM, BM), :] + sqbh_c_ref[...] + g
        kb = jnp.minimum(jnp.exp(arg), 1.0)
        rows = j * BM + jax.lax.broadcasted_iota(jnp.int32, (BM, N), 0)
        cols = jax.lax.broadcasted_iota(jnp.int32, (BM, N), 1)
        kb_bf = jnp.where(rows == cols, 0.0, kb).astype(jnp.bfloat16)
        kb_s[pl.ds(j * BM, BM), :] = kb_bf

        @pl.when(j == 0)
        def _init_q():
            logits = -ut_ref[...]
            m = jnp.max(logits, axis=0, keepdims=True)
            e = jnp.exp(logits - m)
            q0 = e / jnp.sum(e, axis=0, keepdims=True)
            qa_s[...] = q0.astype(jnp.bfloat16)

    @pl.when((t >= 1) & (t % 2 == 1))
    def _odd():
        _mf_step(qa_s, qb_s)

    @pl.when((t >= 2) & (t % 2 == 0))
    def _even():
        _mf_step(qb_s, qa_s)


@functools.partial(jax.jit, static_argnames=())
def kernel(unary, image, w_bilateral, w_spatial):
    h, w, c = unary.shape
    n = h * w
    UT = unary.reshape(n, c).T

    ys, xs = jnp.meshgrid(jnp.arange(h, dtype=jnp.float32),
                          jnp.arange(w, dtype=jnp.float32), indexing="ij")
    coords = jnp.stack([xs, ys], axis=-1).reshape(-1, 2)
    rgb = image.reshape(-1, 3)
    fb = jnp.concatenate([coords / THETA_ALPHA, rgb / THETA_BETA], axis=-1)
    fs = coords / THETA_GAMMA
    zeros3 = jnp.zeros((n, 3), jnp.float32)
    zeros5 = jnp.zeros((n, 5), jnp.float32)
    fbz = jnp.concatenate([fb, zeros3], axis=-1)
    fsz = jnp.concatenate([zeros5, fs, zeros3[:, :1]], axis=-1)
    ft = (fbz + fsz).T
    sqbh = -0.5 * jnp.sum(fb * fb, axis=-1)
    sqsh = -0.5 * jnp.sum(fs * fs, axis=-1)
    wb = jnp.reshape(w_bilateral.astype(jnp.float32), (1,))
    ws = jnp.reshape(w_spatial.astype(jnp.float32), (1,))

    Ks = pl.pallas_call(
        _ks_build_kernel,
        grid=(NB,),
        in_specs=[
            pl.BlockSpec((n, 8), lambda j: (0, 0)),
            pl.BlockSpec((8, n), lambda j: (0, 0)),
            pl.BlockSpec((n, 1), lambda j: (0, 0)),
            pl.BlockSpec((1, n), lambda j: (0, 0)),
        ],
        out_specs=pl.BlockSpec((WIN, BM), lambda j: (0, j)),
        out_shape=jax.ShapeDtypeStruct((WIN, n), jnp.bfloat16),
    )(fsz, ft, sqsh[:, None], sqsh[None, :])

    QT = pl.pallas_call(
        _fused_kernel,
        grid=(N_ITERS + 1, NB),
        in_specs=[
            pl.BlockSpec((n, 8), lambda t, j: (0, 0)),
            pl.BlockSpec((8, n), lambda t, j: (0, 0)),
            pl.BlockSpec((n, 1), lambda t, j: (0, 0)),
            pl.BlockSpec((1, n), lambda t, j: (0, 0)),
            pl.BlockSpec((c, n), lambda t, j: (0, 0)),
            pl.BlockSpec((WIN, BM), lambda t, j: (0, jnp.where(t == 0, 0, j))),
            pl.BlockSpec(memory_space=pltpu.SMEM),
            pl.BlockSpec(memory_space=pltpu.SMEM),
        ],
        out_specs=pl.BlockSpec((c, n), lambda t, j: (0, 0)),
        out_shape=jax.ShapeDtypeStruct((c, n), jnp.float32),
        scratch_shapes=[
            pltpu.VMEM((n, n), jnp.bfloat16),
            pltpu.VMEM((c, n), jnp.bfloat16),
            pltpu.VMEM((c, n), jnp.bfloat16),
        ],
        compiler_params=pltpu.CompilerParams(
            vmem_limit_bytes=63 * 1024 * 1024,
        ),
    )(fbz, ft, sqbh[:, None], sqbh[None, :], UT, Ks, wb, ws)
    return QT.T.reshape(h, w, c)
